# baseline (speedup 1.0000x reference)
import functools

import jax
import jax.numpy as jnp
from jax import lax
from jax.experimental import pallas as pl
from jax.experimental.pallas import tpu as pltpu
from jax.experimental.pallas import tpu_sc as plsc

V = 1_000_000
D = 64
SC = 8.0
B = 4096 * 200
NW = 32
CH_B = 512
NCHUNK = 200 * 8
CH_PER_W = NCHUNK // NW
NPAIR = CH_PER_W // 2

_mesh = plsc.VectorSubcoreMesh(core_axis_name="c", subcore_axis_name="s")


@functools.partial(
    pl.kernel,
    mesh=_mesh,
    compiler_params=pltpu.CompilerParams(
        needs_layout_passes=False, use_tc_tiling_on_sc=False
    ),
    out_type=jax.ShapeDtypeStruct((200, D // 2, 4096), jnp.int32),
    scratch_types=[
        pltpu.VMEM((4, 128), jnp.int32),
        pltpu.VMEM((4, 128), jnp.int32),
        pltpu.VMEM((CH_B, D), jnp.float32),
        pltpu.VMEM((CH_B, D), jnp.float32),
        pltpu.VMEM((D // 2, CH_B), jnp.int32),
        pltpu.VMEM((D // 2, CH_B), jnp.int32),
        pltpu.SemaphoreType.DMA,
        pltpu.SemaphoreType.DMA,
        pltpu.SemaphoreType.DMA,
        pltpu.SemaphoreType.DMA,
    ],
)
def _emb(
    w_hbm, idx_hbm, out_hbm,
    idx_a, idx_b, rows_a, rows_b, out_a, out_b,
    gsem_a, gsem_b, osem_a, osem_b,
):
    wid = lax.axis_index("s") * 2 + lax.axis_index("c")
    iota = lax.iota(jnp.int32, 16)
    bufs = (
        (idx_a, rows_a, out_a, gsem_a, osem_a),
        (idx_b, rows_b, out_b, gsem_b, osem_b),
    )

    def fire(c, slot):
        idx_v, rows_v, _, gsem, _ = bufs[slot]
        q0 = (c // 8) * 32 + (c % 8) * 4
        pltpu.sync_copy(idx_hbm.at[pl.ds(q0, 4)], idx_v)
        for k in range(4):
            pltpu.async_copy(
                w_hbm.at[idx_v.at[k]], rows_v.at[pl.ds(k * 128, 128)], gsem
            )

    def drain_gather(slot):
        _, rows_v, _, gsem, _ = bufs[slot]
        for k in range(4):
            pltpu.make_async_copy(
                w_hbm.at[pl.ds(0, 128)], rows_v.at[pl.ds(k * 128, 128)], gsem
            ).wait()

    def drain_store(slot):
        _, _, out_v, _, osem = bufs[slot]
        pltpu.make_async_copy(
            out_hbm.at[0, pl.ds(0, D // 2), pl.ds(0, CH_B)], out_v, osem
        ).wait()

    def compute_store(c, slot):
        _, rows_v, out_v, _, osem = bufs[slot]

        @plsc.parallel_loop(0, (D // 2) * 32, 1, unroll=8)
        def _vec(q):
            d2 = q >> 5
            bg = q & 31
            rv = bg * 16 + iota
            a = plsc.load_gather(rows_v, [rv, jnp.full((16,), 2 * d2, jnp.int32)])
            b = plsc.load_gather(rows_v, [rv, jnp.full((16,), 2 * d2 + 1, jnp.int32)])
            w = plsc.pack(a * SC, b * SC, format=plsc.PackFormat.INTERLEAVED)
            out_v[d2, pl.ds(bg * 16, 16)] = plsc.bitcast(w, jnp.int32)

        s = c // 8
        b0 = (c % 8) * CH_B
        pltpu.async_copy(
            out_v, out_hbm.at[s, pl.ds(0, D // 2), pl.ds(b0, CH_B)], osem
        )

    c0 = wid * CH_PER_W
    fire(c0, 0)
    fire(c0 + 1, 1)

    def pair_body(p, carry):
        for slot in range(2):
            c = c0 + 2 * p + slot
            drain_gather(slot)
            pl.when(p > 0)(lambda slot=slot: drain_store(slot))
            compute_store(c, slot)
            pl.when(p < NPAIR - 1)(lambda c=c, slot=slot: fire(c + 2, slot))
        return carry

    lax.fori_loop(0, NPAIR, pair_body, 0)
    drain_store(0)
    drain_store(1)


def kernel(input, weight):
    idx2 = input.T.reshape(200 * 32, 128)
    words = _emb(weight, idx2)
    p4 = jax.lax.bitcast_convert_type(words, jnp.bfloat16)
    return p4.transpose(2, 0, 1, 3).reshape(4096, 200, D)


# speedup vs baseline: 1.6017x; 1.6017x over previous
import functools

import jax
import jax.numpy as jnp
from jax import lax
from jax.experimental import pallas as pl
from jax.experimental.pallas import tpu as pltpu
from jax.experimental.pallas import tpu_sc as plsc

V = 1_000_000
D = 64
SC = 8.0
B = 4096 * 200
NW = 32
ROWS_PER_W = B // NW
CHUNK = 256
K = CHUNK // 128
NCHUNK = ROWS_PER_W // CHUNK
NPAIR = NCHUNK // 2

_mesh = plsc.VectorSubcoreMesh(core_axis_name="c", subcore_axis_name="s")


@functools.partial(
    pl.kernel,
    mesh=_mesh,
    compiler_params=pltpu.CompilerParams(
        needs_layout_passes=False, use_tc_tiling_on_sc=False
    ),
    out_type=jax.ShapeDtypeStruct((B, D), jnp.bfloat16),
    scratch_types=[
        pltpu.VMEM((K, 128), jnp.int32),
        pltpu.VMEM((K, 128), jnp.int32),
        pltpu.VMEM((CHUNK, D), jnp.float32),
        pltpu.VMEM((CHUNK, D), jnp.float32),
        pltpu.VMEM((CHUNK, D), jnp.bfloat16),
        pltpu.VMEM((CHUNK, D), jnp.bfloat16),
        pltpu.SemaphoreType.DMA,
        pltpu.SemaphoreType.DMA,
        pltpu.SemaphoreType.DMA,
        pltpu.SemaphoreType.DMA,
    ],
)
def _emb(
    w_hbm, idx_hbm, out_hbm,
    idx_a, idx_b, rows_a, rows_b, out_a, out_b,
    gsem_a, gsem_b, osem_a, osem_b,
):
    wid = lax.axis_index("s") * 2 + lax.axis_index("c")
    iota = lax.iota(jnp.int32, 16)
    ev = iota * 2
    bufs = (
        (idx_a, rows_a, out_a, gsem_a, osem_a),
        (idx_b, rows_b, out_b, gsem_b, osem_b),
    )

    def fire(g, slot):
        idx_v, rows_v, _, gsem, _ = bufs[slot]
        grp0 = wid * (ROWS_PER_W // 128) + g * K
        pltpu.sync_copy(idx_hbm.at[pl.ds(grp0, K)], idx_v)
        for k in range(K):
            pltpu.async_copy(
                w_hbm.at[idx_v.at[k]], rows_v.at[pl.ds(k * 128, 128)], gsem
            )

    def drain_gather(slot):
        _, rows_v, _, gsem, _ = bufs[slot]
        for k in range(K):
            pltpu.make_async_copy(
                w_hbm.at[pl.ds(0, 128)], rows_v.at[pl.ds(k * 128, 128)], gsem
            ).wait()

    def drain_store(slot):
        _, _, out_v, _, osem = bufs[slot]
        pltpu.make_async_copy(
            out_hbm.at[pl.ds(0, CHUNK)], out_v, osem
        ).wait()

    def compute_store(g, slot):
        _, rows_v, out_v, _, osem = bufs[slot]

        @plsc.parallel_loop(0, CHUNK, 1, unroll=8)
        def _row(r):
            re = jnp.full((16,), r, dtype=jnp.int32)
            for h in range(2):
                ce = ev + 32 * h
                a = plsc.load_gather(rows_v, [re, ce])
                b = plsc.load_gather(rows_v, [re, ce + 1])
                out_v[r, pl.ds(32 * h, 32)] = plsc.pack(
                    a * SC, b * SC, format=plsc.PackFormat.INTERLEAVED
                )

        row0 = wid * ROWS_PER_W + g * CHUNK
        pltpu.async_copy(out_v, out_hbm.at[pl.ds(row0, CHUNK)], osem)

    fire(0, 0)
    fire(1, 1)

    def pair_body(p, carry):
        for slot in range(2):
            g = 2 * p + slot
            drain_gather(slot)
            pl.when(p > 0)(lambda slot=slot: drain_store(slot))
            compute_store(g, slot)
            pl.when(p < NPAIR - 1)(lambda g=g, slot=slot: fire(g + 2, slot))
        return carry

    lax.fori_loop(0, NPAIR, pair_body, 0)
    drain_store(0)
    drain_store(1)


def kernel(input, weight):
    idx2 = input.reshape(B // 128, 128)
    return _emb(weight, idx2).reshape(4096, 200, D)
